# Initial kernel scaffold; baseline (speedup 1.0000x reference)
import jax
import jax.numpy as jnp
from jax.experimental import pallas as pl


def kernel(x, pos, pe_w, pe_b, Wq, Wk, Wv, k_intra, v_intra, kc_w, kc_b, vc_w, vc_b, comb_w, comb_b, out_w, out_b):
    raise NotImplementedError("write your pallas kernel here")


# speedup vs baseline: 3.1746x; 3.1746x over previous
import jax
import jax.numpy as jnp
import numpy as np
from jax.experimental import pallas as pl
from jax.experimental.pallas import tpu as pltpu

BS = 1; S = 2048; DIM = 1024; H = 16; KVH = 4; G = H // KVH; DH = DIM // H
SEL = 64; NBLK = S // SEL; TOPK = 2; BALL = 256; NBALL = S // BALL; PDIM = 3
SCALE = 1.0 / np.sqrt(DH)

_HI = jax.lax.Precision.HIGHEST
_TS = 256
_KC = 512
_BF = jnp.bfloat16


def _dot(a, b):
    return jax.lax.dot(a, b, preferred_element_type=jnp.float32)


def _dotT(a, b):
    return jax.lax.dot_general(
        a, b, (((1,), (1,)), ((), ())), preferred_element_type=jnp.float32)


def _proj_body(posp_ref, x_ref, pe_w_ref, pe_b_ref, wq_ref, wk_ref, wv_ref,
               cw_ref, cb_ref,
               k_ref, v_ref, q16_ref, k16_ref, v16_ref, gates_ref):
    p = posp_ref[...]
    ri = jax.lax.broadcasted_iota(jnp.int32, (_TS, _TS), 0)
    ci = jax.lax.broadcasted_iota(jnp.int32, (_TS, _TS), 1)
    avg = jnp.where(ri // SEL == ci // SEL, 1.0 / SEL, 0.0).astype(jnp.float32)
    rel = p - jax.lax.dot(avg, p, precision=_HI)
    xr = (x_ref[...] + _dot(rel.astype(_BF), pe_w_ref[...])
          + pe_b_ref[...])
    x16 = xr.astype(_BF)
    k = _dot(x16, wk_ref[...])
    v = _dot(x16, wv_ref[...])
    k_ref[...] = k
    v_ref[...] = v
    q16_ref[...] = _dot(x16, wq_ref[...]).astype(_BF)
    k16_ref[...] = k.astype(_BF)
    v16_ref[...] = v.astype(_BF)
    gates_ref[...] = jax.nn.sigmoid(_dot(x16, cw_ref[...]) + cb_ref[...])


def _comp_body(kbf_ref, vbf_ref, kif_ref, vif_ref, kcw_ref, kcb_ref,
               vcw_ref, vcb_ref, q16_ref,
               cout_ref, sel_ref):
    kb = (kbf_ref[0] + kif_ref[0]).astype(_BF)
    vb = (vbf_ref[0] + vif_ref[0]).astype(_BF)
    ck = (_dot(kb, kcw_ref[...]) + kcb_ref[...]).astype(_BF)
    cv = (_dot(vb, vcw_ref[...]) + vcb_ref[...]).astype(_BF)
    imp = jnp.zeros((S, NBLK), jnp.float32)
    for g in range(G):
        qh = q16_ref[:, g * DH:(g + 1) * DH]
        sim = _dotT(qh, ck) * SCALE
        m = jnp.max(sim, axis=1, keepdims=True)
        e = jnp.exp(sim - m)
        attn = e / jnp.sum(e, axis=1, keepdims=True)
        imp = imp + attn
        cout_ref[:, g * DH:(g + 1) * DH] = _dot(attn.astype(_BF), cv)
    imp = imp * (1.0 / G)
    col = jax.lax.broadcasted_iota(jnp.int32, (S, NBLK), 1)
    m0 = jnp.max(imp, axis=1, keepdims=True)
    a0 = jnp.min(jnp.where(imp == m0, col, NBLK), axis=1)
    imp2 = jnp.where(col == a0[:, None], -jnp.inf, imp)
    m1 = jnp.max(imp2, axis=1, keepdims=True)
    a1 = jnp.min(jnp.where(imp2 == m1, col, NBLK), axis=1)
    sel_ref[:, 0:1] = a0[:, None]
    sel_ref[:, 1:2] = a1[:, None]


def _attn_body(q16_ref, k16_ref, v16_ref, cout_ref, gates_ref, sel_ref,
               ow_ref, ob_ref, o_ref, acc_ref):
    i = pl.program_id(0)
    gts = gates_ref[...]
    for kv in range(KVH):
        kk = k16_ref[:, kv * DH:(kv + 1) * DH]
        vv = v16_ref[:, kv * DH:(kv + 1) * DH]
        kball = k16_ref[pl.ds(i * BALL, BALL), kv * DH:(kv + 1) * DH]
        vball = v16_ref[pl.ds(i * BALL, BALL), kv * DH:(kv + 1) * DH]
        s0 = sel_ref[:, kv * 128:kv * 128 + 1]
        s1 = sel_ref[:, kv * 128 + 1:kv * 128 + 2]
        for g in range(G):
            h = kv * G + g
            qh = q16_ref[:, h * DH:(h + 1) * DH]
            fm = jnp.full((BALL, 1), -jnp.inf, jnp.float32)
            fl = jnp.zeros((BALL, 1), jnp.float32)
            facc = jnp.zeros((BALL, DH), jnp.float32)
            for c in range(S // _KC):
                kc = kk[c * _KC:(c + 1) * _KC, :]
                vc = vv[c * _KC:(c + 1) * _KC, :]
                simc = _dotT(qh, kc) * SCALE
                blk = (jax.lax.broadcasted_iota(jnp.int32, (BALL, _KC), 1)
                       + c * _KC) // SEL
                msk = (blk == s0) | (blk == s1)
                fs = jnp.where(msk, simc, -1e30)
                mc = jnp.max(fs, axis=1, keepdims=True)
                mnew = jnp.maximum(fm, mc)
                alpha = jnp.exp(fm - mnew)
                fe = jnp.exp(fs - mnew)
                fl = fl * alpha + jnp.sum(fe, axis=1, keepdims=True)
                facc = facc * alpha + _dot(fe.astype(_BF), vc)
                fm = mnew
            fout = facc / fl
            ssim = _dotT(qh, kball) * SCALE
            sm = jnp.max(ssim, axis=1, keepdims=True)
            se = jnp.exp(ssim - sm)
            sattn = se / jnp.sum(se, axis=1, keepdims=True)
            sout = _dot(sattn.astype(_BF), vball)
            g0 = gts[:, 3 * h:3 * h + 1]
            g1 = gts[:, 3 * h + 1:3 * h + 2]
            g2 = gts[:, 3 * h + 2:3 * h + 3]
            coh = cout_ref[:, h * DH:(h + 1) * DH]
            acc_ref[:, h * DH:(h + 1) * DH] = g0 * coh + g1 * fout + g2 * sout
    o_ref[...] = _dot(acc_ref[...].astype(_BF), ow_ref[...]) + ob_ref[...]


def kernel(x, pos, pe_w, pe_b, Wq, Wk, Wv, k_intra, v_intra, kc_w, kc_b,
           vc_w, vc_b, comb_w, comb_b, out_w, out_b):
    f32 = jnp.float32
    posp = jnp.pad(pos, ((0, 0), (0, 8 - PDIM)))
    pe_wp = jnp.pad(pe_w, ((0, 8 - PDIM), (0, 0))).astype(_BF)

    full = lambda shape: pl.BlockSpec(shape, lambda i: (0, 0))
    rows = lambda w: pl.BlockSpec((_TS, w), lambda i: (i, 0))
    k, v, q16, k16, v16, gates = pl.pallas_call(
        _proj_body,
        grid=(S // _TS,),
        in_specs=[rows(8), rows(DIM), full((8, DIM)), full((1, DIM)),
                  full((DIM, H * DH)), full((DIM, KVH * DH)),
                  full((DIM, KVH * DH)), full((DIM, 3 * H)),
                  full((1, 3 * H))],
        out_specs=[rows(KVH * DH), rows(KVH * DH),
                   rows(H * DH), rows(KVH * DH), rows(KVH * DH),
                   rows(3 * H)],
        out_shape=[
            jax.ShapeDtypeStruct((S, KVH * DH), f32),
            jax.ShapeDtypeStruct((S, KVH * DH), f32),
            jax.ShapeDtypeStruct((S, H * DH), _BF),
            jax.ShapeDtypeStruct((S, KVH * DH), _BF),
            jax.ShapeDtypeStruct((S, KVH * DH), _BF),
            jax.ShapeDtypeStruct((S, 3 * H), f32),
        ],
    )(posp, x, pe_wp, pe_b[None, :], Wq.astype(_BF), Wk.astype(_BF),
      Wv.astype(_BF), comb_w.astype(_BF), comb_b[None, :])

    kbf = (k.reshape(NBLK, SEL, KVH, DH).transpose(2, 0, 1, 3)
           .reshape(KVH, NBLK, SEL * DH))
    vbf = (v.reshape(NBLK, SEL, KVH, DH).transpose(2, 0, 1, 3)
           .reshape(KVH, NBLK, SEL * DH))
    kif = k_intra.reshape(KVH, 1, SEL * DH)
    vif = v_intra.reshape(KVH, 1, SEL * DH)

    cout, sel = pl.pallas_call(
        _comp_body,
        grid=(KVH,),
        in_specs=[
            pl.BlockSpec((1, NBLK, SEL * DH), lambda i: (i, 0, 0)),
            pl.BlockSpec((1, NBLK, SEL * DH), lambda i: (i, 0, 0)),
            pl.BlockSpec((1, 1, SEL * DH), lambda i: (i, 0, 0)),
            pl.BlockSpec((1, 1, SEL * DH), lambda i: (i, 0, 0)),
            pl.BlockSpec((SEL * DH, DH), lambda i: (0, 0)),
            pl.BlockSpec((1, DH), lambda i: (0, 0)),
            pl.BlockSpec((SEL * DH, DH), lambda i: (0, 0)),
            pl.BlockSpec((1, DH), lambda i: (0, 0)),
            pl.BlockSpec((S, G * DH), lambda i: (0, i)),
        ],
        out_specs=[
            pl.BlockSpec((S, G * DH), lambda i: (0, i)),
            pl.BlockSpec((S, 128), lambda i: (0, i)),
        ],
        out_shape=[
            jax.ShapeDtypeStruct((S, H * DH), f32),
            jax.ShapeDtypeStruct((S, KVH * 128), jnp.int32),
        ],
    )(kbf, vbf, kif, vif, kc_w.astype(_BF), kc_b[None, :],
      vc_w.astype(_BF), vc_b[None, :], q16)

    ow16 = out_w.astype(_BF)
    ballrows = lambda w: pl.BlockSpec((BALL, w), lambda i: (i, 0))
    o = pl.pallas_call(
        _attn_body,
        grid=(NBALL,),
        in_specs=[
            ballrows(H * DH),
            pl.BlockSpec((S, KVH * DH), lambda i: (0, 0)),
            pl.BlockSpec((S, KVH * DH), lambda i: (0, 0)),
            ballrows(H * DH),
            ballrows(3 * H),
            ballrows(KVH * 128),
            pl.BlockSpec((H * DH, DIM), lambda i: (0, 0)),
            pl.BlockSpec((1, DIM), lambda i: (0, 0)),
        ],
        out_specs=ballrows(DIM),
        out_shape=jax.ShapeDtypeStruct((S, DIM), f32),
        scratch_shapes=[pltpu.VMEM((BALL, H * DH), f32)],
    )(q16, k16, v16, cout, gates, sel, ow16, out_b[None, :])
    return o
